# baseline (speedup 1.0000x reference)
import jax
import jax.numpy as jnp
from jax import lax
from jax.experimental import pallas as pl
from jax.experimental.pallas import tpu as pltpu
from jax.experimental.pallas import tpu_sc as plsc

_B = 4096
_N_SPARSE = 26
_N_DENSE = 13
_VOCAB = 100000
_NC = 2
_NS = 16
_NW = _NC * _NS
_RPW = _B // _NW
_L = 16
_NF = _N_SPARSE + _N_DENSE

_CH = 448
_CPF = 224
_NCHUNK = _N_SPARSE * _CPF
_KPW = _NCHUNK // _NW
_CLAMP = _VOCAB - _CH


def _depad_body(tab_hbm, flat_hbm, stage0_v, stage1_v, cbuf0_v, cbuf1_v,
                isem0, isem1, osem0, osem1):
    wid = lax.axis_index("s") * _NC + lax.axis_index("c")
    lane = lax.iota(jnp.int32, _L)
    zero = jnp.zeros((_L,), jnp.int32)

    def chunk_coords(k):
        g = wid + k * _NW
        f = g // _CPF
        c = g % _CPF
        s = jnp.where(c == _CPF - 1, _CLAMP, c * _CH)
        return f, s

    def in_copy(k, stage, sem):
        f, s = chunk_coords(k)
        return pltpu.make_async_copy(
            tab_hbm.at[f, pl.ds(s, _CH), :], stage, sem)

    def out_copy(k, cbuf, sem):
        f, s = chunk_coords(k)
        return pltpu.make_async_copy(
            cbuf, flat_hbm.at[pl.ds(f * _VOCAB + s, _CH)], sem)

    def compact(stage, cbuf):
        for j in range(_CH // _L):
            row = lane + j * _L
            cbuf[pl.ds(j * _L, _L)] = plsc.load_gather(stage, [row, zero])

    in_copy(0, stage0_v, isem0).start()
    in_copy(1, stage1_v, isem1).start()

    def body(t, _):
        k0 = t * 2
        k1 = k0 + 1

        @pl.when(t > 0)
        def _():
            out_copy(k0 - 2, cbuf0_v, osem0).wait()
        in_copy(k0, stage0_v, isem0).wait()
        compact(stage0_v, cbuf0_v)

        @pl.when(t < _KPW // 2 - 1)
        def _():
            in_copy(k0 + 2, stage0_v, isem0).start()
        out_copy(k0, cbuf0_v, osem0).start()

        @pl.when(t > 0)
        def _():
            out_copy(k1 - 2, cbuf1_v, osem1).wait()
        in_copy(k1, stage1_v, isem1).wait()
        compact(stage1_v, cbuf1_v)

        @pl.when(t < _KPW // 2 - 1)
        def _():
            in_copy(k1 + 2, stage1_v, isem1).start()
        out_copy(k1, cbuf1_v, osem1).start()
        return ()

    lax.fori_loop(0, _KPW // 2, body, (), unroll=False)
    out_copy(_KPW - 2, cbuf0_v, osem0).wait()
    out_copy(_KPW - 1, cbuf1_v, osem1).wait()


def _lookup_body(x_hbm, table_hbm, wd_hbm, out_hbm,
                 x_v, idx_v, rows_v, wd_v, acc_v, sem):
    wid = lax.axis_index("s") * _NC + lax.axis_index("c")
    base = wid * _RPW

    pltpu.sync_copy(x_hbm.at[pl.ds(base * _NF, _RPW * _NF)], x_v)
    pltpu.sync_copy(wd_hbm, wd_v)
    wdv = wd_v[:]

    lane = lax.iota(jnp.int32, _L)
    for j in range(_RPW // _L):
        sl = pl.ds(j * _L, _L)
        rowbase = lane * _NF + (j * _L * _NF)
        acc = jnp.zeros((_L,), jnp.float32)
        for d in range(_N_DENSE):
            acc = acc + plsc.load_gather(
                x_v, [rowbase + (_N_SPARSE + d)]) * wdv[d]
        acc_v[sl] = acc
        for f in range(_N_SPARSE):
            vals = plsc.load_gather(x_v, [rowbase + f])
            idx_v[f, sl] = vals.astype(jnp.int32) + (f * _VOCAB)

    chunk = 13
    for c0 in range(0, _N_SPARSE, chunk):
        copies = [
            pltpu.make_async_copy(table_hbm.at[idx_v.at[f]], rows_v.at[f], sem)
            for f in range(c0, c0 + chunk)
        ]
        for cp in copies:
            cp.start()
        for cp in copies:
            cp.wait()

    for j in range(_RPW // _L):
        sl = pl.ds(j * _L, _L)
        acc = acc_v[sl]
        for f in range(_N_SPARSE):
            acc = acc + rows_v[f, sl]
        acc_v[sl] = acc

    pltpu.sync_copy(acc_v, out_hbm.at[pl.ds(base, _RPW)])


@jax.jit
def _run(x, table3, wd):
    mesh = plsc.VectorSubcoreMesh(core_axis_name="c", subcore_axis_name="s")
    flat = pl.kernel(
        _depad_body,
        out_type=jax.ShapeDtypeStruct((_N_SPARSE * _VOCAB,), jnp.float32),
        mesh=mesh,
        compiler_params=pltpu.CompilerParams(needs_layout_passes=False),
        scratch_types=[
            pltpu.VMEM((_CH, 1), jnp.float32),
            pltpu.VMEM((_CH, 1), jnp.float32),
            pltpu.VMEM((_CH,), jnp.float32),
            pltpu.VMEM((_CH,), jnp.float32),
            pltpu.SemaphoreType.DMA,
            pltpu.SemaphoreType.DMA,
            pltpu.SemaphoreType.DMA,
            pltpu.SemaphoreType.DMA,
        ],
    )(table3)
    return pl.kernel(
        _lookup_body,
        out_type=jax.ShapeDtypeStruct((_B,), jnp.float32),
        mesh=mesh,
        compiler_params=pltpu.CompilerParams(needs_layout_passes=False),
        scratch_types=[
            pltpu.VMEM((_RPW * _NF,), jnp.float32),
            pltpu.VMEM((_N_SPARSE, _RPW), jnp.int32),
            pltpu.VMEM((_N_SPARSE, _RPW), jnp.float32),
            pltpu.VMEM((_L,), jnp.float32),
            pltpu.VMEM((_RPW,), jnp.float32),
            pltpu.SemaphoreType.DMA,
        ],
    )(x, flat, wd)


def kernel(X, W_emb, W_dense):
    X = X.reshape(-1)
    wd = jnp.pad(W_dense[:, 0], (0, _L - _N_DENSE))
    out = _run(X, W_emb, wd)
    return out.reshape(_B, 1)


# speedup vs baseline: 1.6926x; 1.6926x over previous
import jax
import jax.numpy as jnp
from jax import lax
from jax.experimental import pallas as pl
from jax.experimental.pallas import tpu as pltpu
from jax.experimental.pallas import tpu_sc as plsc

_B = 4096
_N_SPARSE = 26
_N_DENSE = 13
_VOCAB = 100000
_NC = 2
_NS = 16
_NW = _NC * _NS
_RPW = _B // _NW
_L = 16
_NF = _N_SPARSE + _N_DENSE
_NG = _RPW // _L


def _lookup_body(x_hbm, tab_hbm, wd_hbm, out_hbm,
                 x_v, idx_v, stage_v, acc_v, wd_v, sem):
    wid = lax.axis_index("s") * _NC + lax.axis_index("c")
    base = wid * _RPW

    pltpu.sync_copy(x_hbm.at[pl.ds(base * _NF, _RPW * _NF)], x_v)
    pltpu.sync_copy(wd_hbm, wd_v)
    wdv = wd_v[:]

    lane = lax.iota(jnp.int32, _L)
    zero = jnp.zeros((_L,), jnp.int32)

    for j in range(_NG):
        sl = pl.ds(j * _L, _L)
        rowbase = lane * _NF + (j * _L * _NF)
        acc = jnp.zeros((_L,), jnp.float32)
        for d in range(_N_DENSE):
            acc = acc + plsc.load_gather(
                x_v, [rowbase + (_N_SPARSE + d)]) * wdv[d]
        acc_v[sl] = acc
        for f in range(_N_SPARSE):
            vals = plsc.load_gather(x_v, [rowbase + f])
            idx_v[f, sl] = vals.astype(jnp.int32)

    def field_step(f, _):
        def group_copies(g):
            vals = idx_v[f, pl.ds(g * _L, _L)]
            return [
                pltpu.make_async_copy(
                    tab_hbm.at[f, pl.ds(vals[l], 1), :],
                    stage_v.at[pl.ds(g * _L + l, 1), :],
                    sem)
                for l in range(_L)
            ]

        pend = []
        for g in range(_NG):
            cps = group_copies(g)
            for cp in cps:
                cp.start()
            pend.append(cps)
            if g >= 2:
                for cp in pend[g - 2]:
                    cp.wait()
        for gg in (_NG - 2, _NG - 1):
            for cp in pend[gg]:
                cp.wait()

        for j in range(_NG):
            got = plsc.load_gather(stage_v, [lane + j * _L, zero])
            sl = pl.ds(j * _L, _L)
            acc_v[sl] = acc_v[sl] + got
        return ()

    lax.fori_loop(0, _N_SPARSE, field_step, (), unroll=False)

    pltpu.sync_copy(acc_v, out_hbm.at[pl.ds(base, _RPW)])


@jax.jit
def _run(x, tab3, wd):
    mesh = plsc.VectorSubcoreMesh(core_axis_name="c", subcore_axis_name="s")
    return pl.kernel(
        _lookup_body,
        out_type=jax.ShapeDtypeStruct((_B,), jnp.float32),
        mesh=mesh,
        compiler_params=pltpu.CompilerParams(needs_layout_passes=False),
        scratch_types=[
            pltpu.VMEM((_RPW * _NF,), jnp.float32),
            pltpu.VMEM((_N_SPARSE, _RPW), jnp.int32),
            pltpu.VMEM((_RPW, 1), jnp.float32),
            pltpu.VMEM((_RPW,), jnp.float32),
            pltpu.VMEM((_L,), jnp.float32),
            pltpu.SemaphoreType.DMA,
        ],
    )(x, tab3, wd)


def kernel(X, W_emb, W_dense):
    X = X.reshape(-1)
    wd = jnp.pad(W_dense[:, 0], (0, _L - _N_DENSE))
    out = _run(X, W_emb, wd)
    return out.reshape(_B, 1)


# speedup vs baseline: 7.1347x; 4.2152x over previous
import jax
import jax.numpy as jnp
from jax import lax
from jax.experimental import pallas as pl
from jax.experimental.pallas import tpu as pltpu
from jax.experimental.pallas import tpu_sc as plsc

_B = 4096
_N_SPARSE = 26
_N_DENSE = 13
_VOCAB = 100000
_NC = 2
_NS = 16
_NW = _NC * _NS
_RPW = _B // _NW
_L = 16
_NF = _N_SPARSE + _N_DENSE


def _sc_body(x_hbm, table_hbm, wd_hbm, out_hbm,
             x_v, idx_v, rows_v, wd_v, acc_v, sem):
    wid = lax.axis_index("s") * _NC + lax.axis_index("c")
    base = wid * _RPW

    pltpu.sync_copy(x_hbm.at[pl.ds(base * _NF, _RPW * _NF)], x_v)
    pltpu.sync_copy(wd_hbm, wd_v)
    wdv = wd_v[:]

    lane = lax.iota(jnp.int32, _L)
    for j in range(_RPW // _L):
        sl = pl.ds(j * _L, _L)
        rowbase = lane * _NF + (j * _L * _NF)
        acc = jnp.zeros((_L,), jnp.float32)
        for d in range(_N_DENSE):
            acc = acc + plsc.load_gather(
                x_v, [rowbase + (_N_SPARSE + d)]) * wdv[d]
        acc_v[sl] = acc
        for f in range(_N_SPARSE):
            vals = plsc.load_gather(x_v, [rowbase + f])
            idx_v[f, sl] = vals.astype(jnp.int32) + (f * _VOCAB)

    copies = [
        pltpu.make_async_copy(table_hbm.at[idx_v.at[f]], rows_v.at[f], sem)
        for f in range(_N_SPARSE)
    ]
    for cp in copies:
        cp.start()
    for cp in copies:
        cp.wait()

    for j in range(_RPW // _L):
        sl = pl.ds(j * _L, _L)
        acc = acc_v[sl]
        for f in range(_N_SPARSE):
            acc = acc + rows_v[f, sl]
        acc_v[sl] = acc

    pltpu.sync_copy(acc_v, out_hbm.at[pl.ds(base, _RPW)])


@jax.jit
def _run(x, table, wd):
    mesh = plsc.VectorSubcoreMesh(core_axis_name="c", subcore_axis_name="s")
    return pl.kernel(
        _sc_body,
        out_type=jax.ShapeDtypeStruct((_B,), jnp.float32),
        mesh=mesh,
        compiler_params=pltpu.CompilerParams(needs_layout_passes=False),
        scratch_types=[
            pltpu.VMEM((_RPW * _NF,), jnp.float32),
            pltpu.VMEM((_N_SPARSE, _RPW), jnp.int32),
            pltpu.VMEM((_N_SPARSE, _RPW), jnp.float32),
            pltpu.VMEM((_L,), jnp.float32),
            pltpu.VMEM((_RPW,), jnp.float32),
            pltpu.SemaphoreType.DMA,
        ],
    )(x, table, wd)


def kernel(X, W_emb, W_dense):
    table = W_emb.reshape(-1)
    X = X.reshape(-1)
    wd = jnp.pad(W_dense[:, 0], (0, _L - _N_DENSE))
    out = _run(X, table, wd)
    return out.reshape(_B, 1)
